# baseline (speedup 1.0000x reference)
import jax
import jax.numpy as jnp
from jax.experimental import pallas as pl
from jax.experimental.pallas import tpu as pltpu

N_ROWS = 32768
D_MODEL = 2048
BLOCK_ROWS = 256
GRID = N_ROWS // BLOCK_ROWS


def _body(mask_blk, mask_full, x_ref, g_ref, b_ref, o_ref):
    x = x_ref[...]
    any_skip = jnp.max(mask_full[...]) > 0
    keep = mask_blk[0, :, :] == 0
    g = g_ref[0, :][None, :]
    b = b_ref[0, :][None, :]
    y = x * g + b
    o_ref[...] = jnp.where(jnp.logical_and(keep, any_skip), y, x)


def kernel(hidden_states, layer_idx, skip_mask, gamma, beta):
    del layer_idx
    mask_i32 = skip_mask.astype(jnp.int32)
    mask_blk = mask_i32.reshape(GRID, BLOCK_ROWS, 1)
    mask_full = mask_i32.reshape(BLOCK_ROWS, GRID)
    out = pl.pallas_call(
        _body,
        grid=(GRID,),
        in_specs=[
            pl.BlockSpec((1, BLOCK_ROWS, 1), lambda i: (i, 0, 0)),
            pl.BlockSpec((BLOCK_ROWS, GRID), lambda i: (0, 0)),
            pl.BlockSpec((BLOCK_ROWS, D_MODEL), lambda i: (i, 0)),
            pl.BlockSpec((1, D_MODEL), lambda i: (0, 0)),
            pl.BlockSpec((1, D_MODEL), lambda i: (0, 0)),
        ],
        out_specs=pl.BlockSpec((BLOCK_ROWS, D_MODEL), lambda i: (i, 0)),
        out_shape=jax.ShapeDtypeStruct((N_ROWS, D_MODEL), jnp.float32),
        compiler_params=pltpu.CompilerParams(
            dimension_semantics=("arbitrary",),
        ),
    )(mask_blk, mask_full, hidden_states, gamma.reshape(1, D_MODEL),
      beta.reshape(1, D_MODEL))
    return (out, skip_mask)


# speedup vs baseline: 1.1621x; 1.1621x over previous
import jax
import jax.numpy as jnp
from jax.experimental import pallas as pl
from jax.experimental.pallas import tpu as pltpu

N_ROWS = 32768
D_MODEL = 2048
BLOCK_ROWS = 512
GRID = N_ROWS // BLOCK_ROWS


def _body(mask_blk, mask_full, x_ref, g_ref, b_ref, o_ref, any_smem):
    @pl.when(pl.program_id(0) == 0)
    def _():
        any_smem[0] = jnp.max(mask_full[...])

    x = x_ref[...]
    any_skip = any_smem[0] > 0
    keep = mask_blk[0, :, :] == 0
    g = g_ref[0, :][None, :]
    b = b_ref[0, :][None, :]
    y = x * g + b
    o_ref[...] = jnp.where(jnp.logical_and(keep, any_skip), y, x)


def kernel(hidden_states, layer_idx, skip_mask, gamma, beta):
    del layer_idx
    mask_i32 = skip_mask.astype(jnp.int32)
    mask_blk = mask_i32.reshape(GRID, BLOCK_ROWS, 1)
    mask_full = mask_i32.reshape(256, N_ROWS // 256)
    out = pl.pallas_call(
        _body,
        grid=(GRID,),
        in_specs=[
            pl.BlockSpec((1, BLOCK_ROWS, 1), lambda i: (i, 0, 0)),
            pl.BlockSpec((256, N_ROWS // 256), lambda i: (0, 0)),
            pl.BlockSpec((BLOCK_ROWS, D_MODEL), lambda i: (i, 0)),
            pl.BlockSpec((1, D_MODEL), lambda i: (0, 0)),
            pl.BlockSpec((1, D_MODEL), lambda i: (0, 0)),
        ],
        out_specs=pl.BlockSpec((BLOCK_ROWS, D_MODEL), lambda i: (i, 0)),
        out_shape=jax.ShapeDtypeStruct((N_ROWS, D_MODEL), jnp.float32),
        scratch_shapes=[pltpu.SMEM((1,), jnp.int32)],
        compiler_params=pltpu.CompilerParams(
            dimension_semantics=("arbitrary",),
        ),
    )(mask_blk, mask_full, hidden_states, gamma.reshape(1, D_MODEL),
      beta.reshape(1, D_MODEL))
    return (out, skip_mask)
